# baseline (speedup 1.0000x reference)
import functools

import jax
import jax.numpy as jnp
from jax import lax
from jax.experimental import pallas as pl
from jax.experimental.pallas import tpu as pltpu
from jax.experimental.pallas import tpu_sc as plsc

N = 10000
E = 320000
D = 128
H = 128
C = 64
L = 4

NC = 2
NS = 16
HH = H // NC
EPT = E // NS
CHUNK = 112
NB = 2
NCHUNK = 180
EPT_PAD = NCHUNK * CHUNK
RPT = 624
LANES = 16


def _sc_propagate4(h0, src3, dst3, w3, dts, zrows):
    mesh = plsc.VectorSubcoreMesh(core_axis_name="c", subcore_axis_name="s")

    @functools.partial(
        pl.kernel,
        mesh=mesh,
        compiler_params=pltpu.CompilerParams(use_tc_tiling_on_sc=False),
        out_type=jax.ShapeDtypeStruct((NC, N, HH), jnp.float32),
        scratch_types=[
            pltpu.VMEM((NCHUNK, CHUNK), jnp.int32),
            pltpu.VMEM((NCHUNK, CHUNK), jnp.int32),
            pltpu.VMEM((NCHUNK, CHUNK), jnp.float32),
            pltpu.VMEM((L, LANES), jnp.float32),
            pltpu.VMEM((CHUNK, HH), jnp.float32),
            pltpu.VMEM((CHUNK, HH), jnp.float32),
            pltpu.VMEM((CHUNK, HH), jnp.float32),
            pltpu.VMEM((CHUNK, HH), jnp.float32),
            pltpu.VMEM_SHARED((N, HH), jnp.float32),
            pltpu.SemaphoreType.DMA,
            pltpu.SemaphoreType.DMA,
            pltpu.SemaphoreType.DMA,
            pltpu.SemaphoreType.DMA,
        ],
    )
    def k(h_hbm, src_hbm, dst_hbm, w_hbm, dts_hbm, z_hbm, h_out,
          src_v, dst_v, w_v, dts_v, ib0, ib1, ob0, ob1, f_sh,
          g0, g1, s0, s1):
        ibufs = (ib0, ib1)
        obufs = (ob0, ob1)
        gsem = (g0, g1)
        ssem = (s0, s1)
        cid = lax.axis_index("c")
        sid = lax.axis_index("s")
        last = sid == NS - 1
        base = sid * RPT

        pltpu.sync_copy(src_hbm.at[sid], src_v)
        pltpu.sync_copy(dst_hbm.at[sid], dst_v)
        pltpu.sync_copy(w_hbm.at[sid], w_v)
        pltpu.sync_copy(dts_hbm, dts_v)

        def zero_f():
            for z in range(5):
                pltpu.sync_copy(z_hbm,
                                f_sh.at[pl.ds(base + z * CHUNK, CHUNK)])

            @pl.when(last)
            def _():
                r = 640 - 5 * CHUNK
                pltpu.sync_copy(z_hbm.at[pl.ds(0, r)],
                                f_sh.at[pl.ds(base + 5 * CHUNK, r)])

            @pl.when(jnp.logical_not(last))
            def _():
                r = RPT - 5 * CHUNK
                pltpu.sync_copy(z_hbm.at[pl.ds(0, r)],
                                f_sh.at[pl.ds(base + 5 * CHUNK, r)])

        def scale(inb, outb, ci):
            def group_body(g, c2):
                wvec = w_v[ci, pl.ds(g * LANES, LANES)]
                for lane in range(LANES):
                    we = wvec[lane]
                    e = g * LANES + lane
                    for j in range(HH // LANES):
                        sl = pl.ds(j * LANES, LANES)
                        outb[e, sl] = inb[e, sl] * we
                return c2

            lax.fori_loop(0, CHUNK // LANES, group_body, 0, unroll=7)

        def propagate(hc):
            for j in range(NB):
                pltpu.async_copy(hc.at[src_v.at[j]], ibufs[j], gsem[j])

            def pipe_body(i, carry):
                for j in range(NB):
                    c = i * NB + j
                    pltpu.make_async_copy(
                        hc.at[src_v.at[c]], ibufs[j], gsem[j]).wait()

                    @pl.when(i > 0)
                    def _(j=j):
                        pltpu.make_async_copy(
                            obufs[j], f_sh.at[dst_v.at[0]], ssem[j]).wait()

                    scale(ibufs[j], obufs[j], c)
                    pltpu.async_copy(
                        obufs[j], f_sh.at[dst_v.at[c]], ssem[j], add=True)

                    @pl.when(c + NB < NCHUNK)
                    def _(j=j, nc=c + NB):
                        pltpu.async_copy(
                            hc.at[src_v.at[nc]], ibufs[j], gsem[j])

                return carry

            lax.fori_loop(0, NCHUNK // NB, pipe_body, 0)
            for j in range(NB):
                pltpu.make_async_copy(
                    obufs[j], f_sh.at[dst_v.at[0]], ssem[j]).wait()

        hc = h_out.at[cid]

        def update(li):
            dtv = dts_v[li, :]

            def do_rows(off, rows):
                pltpu.sync_copy(hc.at[pl.ds(base + off, rows)],
                                ib0.at[pl.ds(0, rows)])
                pltpu.sync_copy(f_sh.at[pl.ds(base + off, rows)],
                                ob0.at[pl.ds(0, rows)])

                def row_body(g, c2):
                    for lane in range(LANES):
                        e = g * LANES + lane
                        for j in range(HH // LANES):
                            sl = pl.ds(j * LANES, LANES)
                            f = jnp.maximum(ob0[e, sl], 0.0)
                            ob0[e, sl] = ib0[e, sl] + f * dtv
                    return c2

                lax.fori_loop(0, rows // LANES, row_body, 0)
                pltpu.sync_copy(ob0.at[pl.ds(0, rows)],
                                hc.at[pl.ds(base + off, rows)])

            def chunk5(z, c2):
                do_rows(z * CHUNK, CHUNK)
                return c2

            lax.fori_loop(0, 5, chunk5, 0)

            @pl.when(last)
            def _():
                do_rows(5 * CHUNK, 640 - 5 * CHUNK)

            @pl.when(jnp.logical_not(last))
            def _():
                do_rows(5 * CHUNK, RPT - 5 * CHUNK)

        for z in range(5):
            pltpu.sync_copy(h_hbm.at[cid, pl.ds(base + z * CHUNK, CHUNK)],
                            hc.at[pl.ds(base + z * CHUNK, CHUNK)])

        @pl.when(last)
        def _():
            r = 640 - 5 * CHUNK
            pltpu.sync_copy(h_hbm.at[cid, pl.ds(base + 5 * CHUNK, r)],
                            hc.at[pl.ds(base + 5 * CHUNK, r)])

        @pl.when(jnp.logical_not(last))
        def _():
            r = RPT - 5 * CHUNK
            pltpu.sync_copy(h_hbm.at[cid, pl.ds(base + 5 * CHUNK, r)],
                            hc.at[pl.ds(base + 5 * CHUNK, r)])

        zero_f()
        plsc.subcore_barrier()

        def layer_body(li, carry):
            propagate(hc)
            plsc.subcore_barrier()
            update(li)
            zero_f()
            plsc.subcore_barrier()
            return carry

        lax.fori_loop(0, L, layer_body, 0)

    return k(h0, src3, dst3, w3, dts, zrows)


def _tc_entry(x, w1t_s, b1_s):
    def body(x_ref, w_ref, b_ref, o_ref):
        acc = jnp.dot(x_ref[...], w_ref[0],
                      preferred_element_type=jnp.float32)
        o_ref[0] = jnp.maximum(acc + b_ref[0], 0.0)

    return pl.pallas_call(
        body,
        grid=(10, NC),
        in_specs=[
            pl.BlockSpec((N // 10, D), lambda i, p: (i, 0)),
            pl.BlockSpec((1, D, HH), lambda i, p: (p, 0, 0)),
            pl.BlockSpec((1, 1, HH), lambda i, p: (p, 0, 0)),
        ],
        out_specs=pl.BlockSpec((1, N // 10, HH), lambda i, p: (p, i, 0)),
        out_shape=jax.ShapeDtypeStruct((NC, N, HH), jnp.float32),
    )(x, w1t_s, b1_s)


def _tc_final(h_s, w2t, b2):
    def body(h_ref, w_ref, b_ref, o_ref):
        flat = jnp.concatenate([h_ref[0], h_ref[1]], axis=1)
        acc = jnp.dot(flat, w_ref[...], preferred_element_type=jnp.float32)
        o_ref[...] = acc + b_ref[...]

    return pl.pallas_call(
        body,
        grid=(10,),
        in_specs=[
            pl.BlockSpec((NC, N // 10, HH), lambda i: (0, i, 0)),
            pl.BlockSpec((H, C), lambda i: (0, 0)),
            pl.BlockSpec((1, C), lambda i: (0, 0)),
        ],
        out_specs=pl.BlockSpec((N // 10, C), lambda i: (i, 0)),
        out_shape=jax.ShapeDtypeStruct((N, C), jnp.float32),
    )(h_s, w2t, b2.reshape(1, C))


def kernel(x, edge_index, edge_weight, W1, b1, W2, b2, time_step_list):
    pad = ((0, 0), (0, EPT_PAD - EPT))
    dst3 = jnp.pad(edge_index[0].reshape(NS, EPT), pad).reshape(
        NS, NCHUNK, CHUNK)
    src3 = jnp.pad(edge_index[1].reshape(NS, EPT), pad).reshape(
        NS, NCHUNK, CHUNK)
    w3 = jnp.pad(edge_weight.reshape(NS, EPT), pad).reshape(
        NS, NCHUNK, CHUNK)
    dts = jnp.broadcast_to(time_step_list[:, None], (L, LANES))
    zrows = jnp.zeros((CHUNK, HH), jnp.float32)

    w1t_s = W1.T.reshape(D, NC, HH).transpose(1, 0, 2)
    b1_s = b1.reshape(1, NC, HH).transpose(1, 0, 2)
    h = _tc_entry(x, w1t_s, b1_s)
    h = _sc_propagate4(h, src3, dst3, w3, dts, zrows)
    return _tc_final(h, W2.T, b2)


# speedup vs baseline: 1.0671x; 1.0671x over previous
import functools

import jax
import jax.numpy as jnp
from jax import lax
from jax.experimental import pallas as pl
from jax.experimental.pallas import tpu as pltpu
from jax.experimental.pallas import tpu_sc as plsc

N = 10000
E = 320000
D = 128
H = 128
C = 64
L = 4

NC = 2
NS = 16
HH = H // NC
EPT = E // NS
CHUNK = 112
NB = 2
NCHUNK = 180
EPT_PAD = NCHUNK * CHUNK
RPT = 624
LANES = 16


def _sc_propagate4(h0, src3, dst3, w3, dts, zrows):
    mesh = plsc.VectorSubcoreMesh(core_axis_name="c", subcore_axis_name="s")

    @functools.partial(
        pl.kernel,
        mesh=mesh,
        compiler_params=pltpu.CompilerParams(use_tc_tiling_on_sc=False),
        out_type=jax.ShapeDtypeStruct((NC, N, HH), jnp.float32),
        scratch_types=[
            pltpu.VMEM((NCHUNK, CHUNK), jnp.int32),
            pltpu.VMEM((NCHUNK, CHUNK), jnp.int32),
            pltpu.VMEM((NCHUNK, CHUNK), jnp.float32),
            pltpu.VMEM((L, LANES), jnp.float32),
            pltpu.VMEM((CHUNK, HH), jnp.float32),
            pltpu.VMEM((CHUNK, HH), jnp.float32),
            pltpu.VMEM((CHUNK, HH), jnp.float32),
            pltpu.VMEM((CHUNK, HH), jnp.float32),
            pltpu.VMEM_SHARED((N, HH), jnp.float32),
            pltpu.SemaphoreType.DMA,
            pltpu.SemaphoreType.DMA,
            pltpu.SemaphoreType.DMA,
            pltpu.SemaphoreType.DMA,
            pltpu.SemaphoreType.DMA,
            pltpu.SemaphoreType.DMA,
            pltpu.SemaphoreType.DMA,
            pltpu.SemaphoreType.DMA,
        ],
    )
    def k(h_hbm, src_hbm, dst_hbm, w_hbm, dts_hbm, z_hbm, h_out,
          src_v, dst_v, w_v, dts_v, ib0, ib1, ob0, ob1, f_sh,
          g0, g1, s0, s1, fl0, fl1, hw0, hw1):
        ibufs = (ib0, ib1)
        obufs = (ob0, ob1)
        gsem = (g0, g1)
        ssem = (s0, s1)
        flsem = (fl0, fl1)
        hwsem = (hw0, hw1)
        cid = lax.axis_index("c")
        sid = lax.axis_index("s")
        last = sid == NS - 1
        base = sid * RPT

        pltpu.sync_copy(src_hbm.at[sid], src_v)
        pltpu.sync_copy(dst_hbm.at[sid], dst_v)
        pltpu.sync_copy(w_hbm.at[sid], w_v)
        pltpu.sync_copy(dts_hbm, dts_v)

        def zero_f():
            for z in range(5):
                pltpu.sync_copy(z_hbm,
                                f_sh.at[pl.ds(base + z * CHUNK, CHUNK)])

            @pl.when(last)
            def _():
                r = 640 - 5 * CHUNK
                pltpu.sync_copy(z_hbm.at[pl.ds(0, r)],
                                f_sh.at[pl.ds(base + 5 * CHUNK, r)])

            @pl.when(jnp.logical_not(last))
            def _():
                r = RPT - 5 * CHUNK
                pltpu.sync_copy(z_hbm.at[pl.ds(0, r)],
                                f_sh.at[pl.ds(base + 5 * CHUNK, r)])

        def scale(inb, outb, ci):
            def group_body(g, c2):
                wvec = w_v[ci, pl.ds(g * LANES, LANES)]
                for lane in range(LANES):
                    we = wvec[lane]
                    e = g * LANES + lane
                    for j in range(HH // LANES):
                        sl = pl.ds(j * LANES, LANES)
                        outb[e, sl] = inb[e, sl] * we
                return c2

            lax.fori_loop(0, CHUNK // LANES, group_body, 0, unroll=7)

        def propagate(hc):
            for j in range(NB):
                pltpu.async_copy(hc.at[src_v.at[j]], ibufs[j], gsem[j])

            def pipe_body(i, carry):
                for j in range(NB):
                    c = i * NB + j
                    pltpu.make_async_copy(
                        hc.at[src_v.at[c]], ibufs[j], gsem[j]).wait()

                    @pl.when(i > 0)
                    def _(j=j):
                        pltpu.make_async_copy(
                            obufs[j], f_sh.at[dst_v.at[0]], ssem[j]).wait()

                    scale(ibufs[j], obufs[j], c)
                    pltpu.async_copy(
                        obufs[j], f_sh.at[dst_v.at[c]], ssem[j], add=True)

                    @pl.when(c + NB < NCHUNK)
                    def _(j=j, nc=c + NB):
                        pltpu.async_copy(
                            hc.at[src_v.at[nc]], ibufs[j], gsem[j])

                return carry

            lax.fori_loop(0, NCHUNK // NB, pipe_body, 0)
            for j in range(NB):
                pltpu.make_async_copy(
                    obufs[j], f_sh.at[dst_v.at[0]], ssem[j]).wait()

        hc = h_out.at[cid]

        def update(li):
            dtv = dts_v[li, :]

            zeros = jnp.zeros((LANES,), jnp.float32)

            def issue_loads(z, j):
                off = base + z * CHUNK
                pltpu.async_copy(hc.at[pl.ds(off, CHUNK)],
                                 ibufs[j], gsem[j])
                pltpu.async_copy(f_sh.at[pl.ds(off, CHUNK)],
                                 obufs[j], flsem[j])

            def wait_loads(z, j):
                off = base + z * CHUNK
                pltpu.make_async_copy(hc.at[pl.ds(off, CHUNK)],
                                      ibufs[j], gsem[j]).wait()
                pltpu.make_async_copy(f_sh.at[pl.ds(off, CHUNK)],
                                      obufs[j], flsem[j]).wait()

            def issue_stores(z, j):
                off = base + z * CHUNK
                pltpu.async_copy(ibufs[j],
                                 hc.at[pl.ds(off, CHUNK)], hwsem[j])
                pltpu.async_copy(obufs[j],
                                 f_sh.at[pl.ds(off, CHUNK)], ssem[j])

            def wait_stores(z, j):
                off = base + z * CHUNK
                pltpu.make_async_copy(ibufs[j],
                                      hc.at[pl.ds(off, CHUNK)],
                                      hwsem[j]).wait()
                pltpu.make_async_copy(obufs[j],
                                      f_sh.at[pl.ds(off, CHUNK)],
                                      ssem[j]).wait()

            def compute(j, rows):
                inb, fb = ibufs[j], obufs[j]

                def row_body(g, c2):
                    for lane in range(LANES):
                        e = g * LANES + lane
                        for jj in range(HH // LANES):
                            sl = pl.ds(jj * LANES, LANES)
                            f = jnp.maximum(fb[e, sl], 0.0)
                            inb[e, sl] = inb[e, sl] + f * dtv
                            fb[e, sl] = zeros
                    return c2

                lax.fori_loop(0, rows // LANES, row_body, 0)

            issue_loads(0, 0)
            issue_loads(1, 1)
            for z in range(5):
                j = z % 2
                wait_loads(z, j)
                compute(j, CHUNK)
                issue_stores(z, j)
                if z + 2 <= 4:
                    wait_stores(z, j)
                    issue_loads(z + 2, j)

            wait_stores(3, 1)

            def do_tail(rows):
                off = base + 5 * CHUNK
                pltpu.async_copy(hc.at[pl.ds(off, rows)],
                                 ibufs[1].at[pl.ds(0, rows)], gsem[1])
                pltpu.async_copy(f_sh.at[pl.ds(off, rows)],
                                 obufs[1].at[pl.ds(0, rows)], flsem[1])
                pltpu.make_async_copy(hc.at[pl.ds(off, rows)],
                                      ibufs[1].at[pl.ds(0, rows)],
                                      gsem[1]).wait()
                pltpu.make_async_copy(f_sh.at[pl.ds(off, rows)],
                                      obufs[1].at[pl.ds(0, rows)],
                                      flsem[1]).wait()
                compute(1, rows)
                pltpu.async_copy(ibufs[1].at[pl.ds(0, rows)],
                                 hc.at[pl.ds(off, rows)], hwsem[1])
                pltpu.async_copy(obufs[1].at[pl.ds(0, rows)],
                                 f_sh.at[pl.ds(off, rows)], ssem[1])
                pltpu.make_async_copy(ibufs[1].at[pl.ds(0, rows)],
                                      hc.at[pl.ds(off, rows)],
                                      hwsem[1]).wait()
                pltpu.make_async_copy(obufs[1].at[pl.ds(0, rows)],
                                      f_sh.at[pl.ds(off, rows)],
                                      ssem[1]).wait()

            @pl.when(last)
            def _():
                do_tail(640 - 5 * CHUNK)

            @pl.when(jnp.logical_not(last))
            def _():
                do_tail(RPT - 5 * CHUNK)

            wait_stores(4, 0)

        for z in range(5):
            pltpu.sync_copy(h_hbm.at[cid, pl.ds(base + z * CHUNK, CHUNK)],
                            hc.at[pl.ds(base + z * CHUNK, CHUNK)])

        @pl.when(last)
        def _():
            r = 640 - 5 * CHUNK
            pltpu.sync_copy(h_hbm.at[cid, pl.ds(base + 5 * CHUNK, r)],
                            hc.at[pl.ds(base + 5 * CHUNK, r)])

        @pl.when(jnp.logical_not(last))
        def _():
            r = RPT - 5 * CHUNK
            pltpu.sync_copy(h_hbm.at[cid, pl.ds(base + 5 * CHUNK, r)],
                            hc.at[pl.ds(base + 5 * CHUNK, r)])

        zero_f()
        plsc.subcore_barrier()

        def layer_body(li, carry):
            propagate(hc)
            plsc.subcore_barrier()
            update(li)
            plsc.subcore_barrier()
            return carry

        lax.fori_loop(0, L, layer_body, 0)

    return k(h0, src3, dst3, w3, dts, zrows)


def _tc_entry(x, w1t_s, b1_s):
    def body(x_ref, w_ref, b_ref, o_ref):
        acc = jnp.dot(x_ref[...], w_ref[0],
                      preferred_element_type=jnp.float32)
        o_ref[0] = jnp.maximum(acc + b_ref[0], 0.0)

    return pl.pallas_call(
        body,
        grid=(10, NC),
        in_specs=[
            pl.BlockSpec((N // 10, D), lambda i, p: (i, 0)),
            pl.BlockSpec((1, D, HH), lambda i, p: (p, 0, 0)),
            pl.BlockSpec((1, 1, HH), lambda i, p: (p, 0, 0)),
        ],
        out_specs=pl.BlockSpec((1, N // 10, HH), lambda i, p: (p, i, 0)),
        out_shape=jax.ShapeDtypeStruct((NC, N, HH), jnp.float32),
    )(x, w1t_s, b1_s)


def _tc_final(h_s, w2t, b2):
    def body(h_ref, w_ref, b_ref, o_ref):
        flat = jnp.concatenate([h_ref[0], h_ref[1]], axis=1)
        acc = jnp.dot(flat, w_ref[...], preferred_element_type=jnp.float32)
        o_ref[...] = acc + b_ref[...]

    return pl.pallas_call(
        body,
        grid=(10,),
        in_specs=[
            pl.BlockSpec((NC, N // 10, HH), lambda i: (0, i, 0)),
            pl.BlockSpec((H, C), lambda i: (0, 0)),
            pl.BlockSpec((1, C), lambda i: (0, 0)),
        ],
        out_specs=pl.BlockSpec((N // 10, C), lambda i: (i, 0)),
        out_shape=jax.ShapeDtypeStruct((N, C), jnp.float32),
    )(h_s, w2t, b2.reshape(1, C))


def kernel(x, edge_index, edge_weight, W1, b1, W2, b2, time_step_list):
    pad = ((0, 0), (0, EPT_PAD - EPT))
    dst3 = jnp.pad(edge_index[0].reshape(NS, EPT), pad).reshape(
        NS, NCHUNK, CHUNK)
    src3 = jnp.pad(edge_index[1].reshape(NS, EPT), pad).reshape(
        NS, NCHUNK, CHUNK)
    w3 = jnp.pad(edge_weight.reshape(NS, EPT), pad).reshape(
        NS, NCHUNK, CHUNK)
    dts = jnp.broadcast_to(time_step_list[:, None], (L, LANES))
    zrows = jnp.zeros((CHUNK, HH), jnp.float32)

    w1t_s = W1.T.reshape(D, NC, HH).transpose(1, 0, 2)
    b1_s = b1.reshape(1, NC, HH).transpose(1, 0, 2)
    h = _tc_entry(x, w1t_s, b1_s)
    h = _sc_propagate4(h, src3, dst3, w3, dts, zrows)
    return _tc_final(h, W2.T, b2)


# speedup vs baseline: 1.1216x; 1.0511x over previous
import functools

import jax
import jax.numpy as jnp
from jax import lax
from jax.experimental import pallas as pl
from jax.experimental.pallas import tpu as pltpu
from jax.experimental.pallas import tpu_sc as plsc

N = 10000
E = 320000
D = 128
H = 128
C = 64
L = 4

NC = 2
NS = 16
HH = H // NC
EPT = E // NS
CHUNK = 112
NB = 2
NCHUNK = 180
EPT_PAD = NCHUNK * CHUNK
RPT = 624
LANES = 16


def _sc_propagate(h_s, src3, dst3, w3):
    mesh = plsc.VectorSubcoreMesh(core_axis_name="c", subcore_axis_name="s")

    @functools.partial(
        pl.kernel,
        mesh=mesh,
        compiler_params=pltpu.CompilerParams(use_tc_tiling_on_sc=False),
        out_type=jax.ShapeDtypeStruct((NC, N, HH), jnp.float32),
        scratch_types=[
            pltpu.VMEM((NCHUNK, CHUNK), jnp.int32),
            pltpu.VMEM((NCHUNK, CHUNK), jnp.int32),
            pltpu.VMEM((NCHUNK, CHUNK), jnp.float32),
            pltpu.VMEM((CHUNK, HH), jnp.float32),
            pltpu.VMEM((CHUNK, HH), jnp.float32),
            pltpu.VMEM((CHUNK, HH), jnp.float32),
            pltpu.VMEM((CHUNK, HH), jnp.float32),
            pltpu.VMEM_SHARED((N, HH), jnp.float32),
            pltpu.SemaphoreType.DMA,
            pltpu.SemaphoreType.DMA,
            pltpu.SemaphoreType.DMA,
            pltpu.SemaphoreType.DMA,
        ],
    )
    def k(h_hbm, src_hbm, dst_hbm, w_hbm, f_out, src_v, dst_v, w_v,
          ib0, ib1, ob0, ob1, f_sh, g0, g1, s0, s1):
        ibufs = (ib0, ib1)
        obufs = (ob0, ob1)
        gsem = (g0, g1)
        ssem = (s0, s1)
        cid = lax.axis_index("c")
        sid = lax.axis_index("s")
        last = sid == NS - 1
        hc = h_hbm.at[cid]

        pltpu.sync_copy(src_hbm.at[sid], src_v)
        pltpu.sync_copy(dst_hbm.at[sid], dst_v)
        pltpu.sync_copy(w_hbm.at[sid], w_v)

        zeros = jnp.zeros((LANES,), jnp.float32)

        def zrow(r, carry):
            for j in range(HH // LANES):
                ob0[r, pl.ds(j * LANES, LANES)] = zeros
            return carry

        lax.fori_loop(0, CHUNK, zrow, 0)
        base = sid * RPT
        for z in range(5):
            pltpu.sync_copy(ob0, f_sh.at[pl.ds(base + z * CHUNK, CHUNK)])

        @pl.when(last)
        def _():
            pltpu.sync_copy(ob0.at[pl.ds(0, 640 - 5 * CHUNK)],
                            f_sh.at[pl.ds(base + 5 * CHUNK, 640 - 5 * CHUNK)])

        @pl.when(jnp.logical_not(last))
        def _():
            pltpu.sync_copy(ob0.at[pl.ds(0, RPT - 5 * CHUNK)],
                            f_sh.at[pl.ds(base + 5 * CHUNK, RPT - 5 * CHUNK)])

        plsc.subcore_barrier()

        def scale(inb, outb, ci):
            def group_body(g, c2):
                wvec = w_v[ci, pl.ds(g * LANES, LANES)]
                for lane in range(LANES):
                    we = wvec[lane]
                    e = g * LANES + lane
                    for j in range(HH // LANES):
                        sl = pl.ds(j * LANES, LANES)
                        outb[e, sl] = inb[e, sl] * we
                return c2

            lax.fori_loop(0, CHUNK // LANES, group_body, 0)

        for j in range(NB):
            pltpu.async_copy(hc.at[src_v.at[j]], ibufs[j], gsem[j])

        def pipe_body(i, carry):
            for j in range(NB):
                c = i * NB + j
                pltpu.make_async_copy(
                    hc.at[src_v.at[c]], ibufs[j], gsem[j]).wait()

                @pl.when(i > 0)
                def _(j=j):
                    pltpu.make_async_copy(
                        obufs[j], f_sh.at[dst_v.at[0]], ssem[j]).wait()

                scale(ibufs[j], obufs[j], c)
                pltpu.async_copy(
                    obufs[j], f_sh.at[dst_v.at[c]], ssem[j], add=True)

                @pl.when(c + NB < NCHUNK)
                def _(j=j, nc=c + NB):
                    pltpu.async_copy(hc.at[src_v.at[nc]], ibufs[j], gsem[j])

            return carry

        lax.fori_loop(0, NCHUNK // NB, pipe_body, 0)
        for j in range(NB):
            pltpu.make_async_copy(
                obufs[j], f_sh.at[dst_v.at[0]], ssem[j]).wait()
        plsc.subcore_barrier()

        for z in range(5):
            pltpu.sync_copy(f_sh.at[pl.ds(base + z * CHUNK, CHUNK)],
                            f_out.at[cid, pl.ds(base + z * CHUNK, CHUNK)])

        @pl.when(last)
        def _():
            r = 640 - 5 * CHUNK
            pltpu.sync_copy(f_sh.at[pl.ds(base + 5 * CHUNK, r)],
                            f_out.at[cid, pl.ds(base + 5 * CHUNK, r)])

        @pl.when(jnp.logical_not(last))
        def _():
            r = RPT - 5 * CHUNK
            pltpu.sync_copy(f_sh.at[pl.ds(base + 5 * CHUNK, r)],
                            f_out.at[cid, pl.ds(base + 5 * CHUNK, r)])

    return k(h_s, src3, dst3, w3)


def _tc_entry(x, w1t_s, b1_s):
    def body(x_ref, w_ref, b_ref, o_ref):
        acc = jnp.dot(x_ref[...], w_ref[0],
                      preferred_element_type=jnp.float32)
        o_ref[0] = jnp.maximum(acc + b_ref[0], 0.0)

    return pl.pallas_call(
        body,
        grid=(10, NC),
        in_specs=[
            pl.BlockSpec((N // 10, D), lambda i, p: (i, 0)),
            pl.BlockSpec((1, D, HH), lambda i, p: (p, 0, 0)),
            pl.BlockSpec((1, 1, HH), lambda i, p: (p, 0, 0)),
        ],
        out_specs=pl.BlockSpec((1, N // 10, HH), lambda i, p: (p, i, 0)),
        out_shape=jax.ShapeDtypeStruct((NC, N, HH), jnp.float32),
    )(x, w1t_s, b1_s)


def _tc_combine(h_s, f, dt):
    def body(h_ref, f_ref, dt_ref, o_ref):
        o_ref[...] = h_ref[...] + jnp.maximum(f_ref[...], 0.0) * dt_ref[0]

    blk = pl.BlockSpec((1, N // 10, HH), lambda i, p: (p, i, 0))
    return pl.pallas_call(
        body,
        grid=(10, NC),
        in_specs=[blk, blk, pl.BlockSpec(memory_space=pltpu.SMEM)],
        out_specs=blk,
        out_shape=jax.ShapeDtypeStruct((NC, N, HH), jnp.float32),
    )(h_s, f, dt)


def _tc_final(h_s, f, dt, w2t, b2):
    def body(h_ref, f_ref, dt_ref, w_ref, b_ref, o_ref):
        hh = h_ref[...] + jnp.maximum(f_ref[...], 0.0) * dt_ref[0]
        flat = jnp.concatenate([hh[0], hh[1]], axis=1)
        acc = jnp.dot(flat, w_ref[...], preferred_element_type=jnp.float32)
        o_ref[...] = acc + b_ref[...]

    blk = pl.BlockSpec((NC, N // 10, HH), lambda i: (0, i, 0))
    return pl.pallas_call(
        body,
        grid=(10,),
        in_specs=[
            blk, blk,
            pl.BlockSpec(memory_space=pltpu.SMEM),
            pl.BlockSpec((H, C), lambda i: (0, 0)),
            pl.BlockSpec((1, C), lambda i: (0, 0)),
        ],
        out_specs=pl.BlockSpec((N // 10, C), lambda i: (i, 0)),
        out_shape=jax.ShapeDtypeStruct((N, C), jnp.float32),
    )(h_s, f, dt, w2t, b2.reshape(1, C))


def kernel(x, edge_index, edge_weight, W1, b1, W2, b2, time_step_list):
    pad = ((0, 0), (0, EPT_PAD - EPT))
    dst3 = jnp.pad(edge_index[0].reshape(NS, EPT), pad).reshape(
        NS, NCHUNK, CHUNK)
    src3 = jnp.pad(edge_index[1].reshape(NS, EPT), pad).reshape(
        NS, NCHUNK, CHUNK)
    w3 = jnp.pad(edge_weight.reshape(NS, EPT), pad).reshape(
        NS, NCHUNK, CHUNK)

    w1t_s = W1.T.reshape(D, NC, HH).transpose(1, 0, 2)
    b1_s = b1.reshape(1, NC, HH).transpose(1, 0, 2)
    h = _tc_entry(x, w1t_s, b1_s)
    out = None
    for i in range(L):
        f = _sc_propagate(h, src3, dst3, w3)
        dt = time_step_list[i].reshape(1)
        if i < L - 1:
            h = _tc_combine(h, f, dt)
        else:
            out = _tc_final(h, f, dt, W2.T, b2)
    return out
